# Initial kernel scaffold; baseline (speedup 1.0000x reference)
import functools
import math

import jax
import jax.numpy as jnp
from jax import lax
from jax.experimental import pallas as pl
from jax.experimental.pallas import tpu as pltpu
from jax.experimental.pallas import tpu_sc as plsc

N = 10000
E = 320000
D = 128
C = 2
BN_EPS = 1e-5
ISQ = float(1.0 / math.sqrt(1.0 + BN_EPS))

NC = 2
NS = 16
NW = NC * NS
EPW = E // NW
CH = 80
NCHUNK = EPW // CH
RPT = N // NS
DEGW = 16


def _make_seg_sum(with_deg):
  mesh = plsc.VectorSubcoreMesh(core_axis_name="c", subcore_axis_name="s")
  out_type = [jax.ShapeDtypeStruct((NC, N, D), jnp.float32)]
  scratch = [
      pltpu.VMEM((CH,), jnp.int32),
      pltpu.VMEM((CH,), jnp.int32),
      pltpu.VMEM((CH, D), jnp.float32),
      pltpu.VMEM_SHARED((N, D), jnp.float32),
      pltpu.SemaphoreType.DMA,
  ]
  if with_deg:
    out_type.append(jax.ShapeDtypeStruct((NC, N, DEGW), jnp.float32))
    scratch += [
        pltpu.VMEM((CH, DEGW), jnp.float32),
        pltpu.VMEM_SHARED((N, DEGW), jnp.float32),
    ]

  def body(*refs):
    if with_deg:
      (h_hbm, src_hbm, dst_hbm, z_hbm, zd_hbm,
       out_hbm, deg_hbm,
       src_v, dst_v, rows_v, acc, sem, ones_v, dacc) = refs
    else:
      (h_hbm, src_hbm, dst_hbm, z_hbm,
       out_hbm,
       src_v, dst_v, rows_v, acc, sem) = refs

    c = lax.axis_index("c")
    s = lax.axis_index("s")
    wid = s * NC + c
    r0 = s * RPT

    pltpu.sync_copy(z_hbm.at[pl.ds(r0, RPT)], acc.at[pl.ds(r0, RPT)])
    if with_deg:
      pltpu.sync_copy(zd_hbm.at[pl.ds(r0, RPT)], dacc.at[pl.ds(r0, RPT)])
      one16 = jnp.ones((16,), jnp.float32)
      def ofill(i, _):
        ones_v[i, pl.ds(0, DEGW)] = one16
        return 0
      lax.fori_loop(0, CH, ofill, 0)
    plsc.subcore_barrier()

    base = wid * EPW

    def chunk(i, _):
      off = base + i * CH
      pltpu.sync_copy(src_hbm.at[pl.ds(off, CH)], src_v)
      pltpu.sync_copy(dst_hbm.at[pl.ds(off, CH)], dst_v)
      pltpu.async_copy(h_hbm.at[src_v], rows_v, sem).wait()
      pltpu.sync_copy(rows_v, acc.at[dst_v], add=True)
      if with_deg:
        pltpu.sync_copy(ones_v, dacc.at[dst_v], add=True)
      return 0

    lax.fori_loop(0, NCHUNK, chunk, 0)
    plsc.subcore_barrier()

    pltpu.sync_copy(acc.at[pl.ds(r0, RPT)], out_hbm.at[c, pl.ds(r0, RPT)])
    if with_deg:
      pltpu.sync_copy(dacc.at[pl.ds(r0, RPT)], deg_hbm.at[c, pl.ds(r0, RPT)])

  return pl.kernel(
      body,
      out_type=tuple(out_type) if with_deg else out_type[0],
      mesh=mesh,
      scratch_types=scratch,
  )


_seg_sum = _make_seg_sum(with_deg=False)
_seg_sum_deg = _make_seg_sum(with_deg=True)


R = 2000
GRID = N // R

def _row_spec(w):
  return pl.BlockSpec((R, w), lambda i: (i, 0))

def _pair_spec(w):
  return pl.BlockSpec((NC, R, w), lambda i: (0, i, 0))

def _w_spec():
  return pl.BlockSpec((D, D), lambda i: (0, 0))

def _v_spec():
  return pl.BlockSpec((1, D), lambda i: (0, 0))


def _in_proj_body(x_ref, w_ref, b_ref, o_ref):
  o_ref[...] = jnp.dot(x_ref[...], w_ref[...],
                       preferred_element_type=jnp.float32) + b_ref[...]


def _in_proj(x, w, b):
  return pl.pallas_call(
      _in_proj_body,
      grid=(GRID,),
      in_specs=[_row_spec(D), _w_spec(), _v_spec()],
      out_specs=_row_spec(D),
      out_shape=jax.ShapeDtypeStruct((N, D), jnp.float32),
  )(x, w, b)


def _gin_body(h_ref, p_ref, eps_ref, w1_ref, a1_ref, c1_ref,
              w2_ref, a2_ref, c2_ref, a3_ref, c3_ref, o_ref, *, residual):
  h = h_ref[...]
  t = (1.0 + eps_ref[0, 0]) * h + p_ref[0] + p_ref[1]
  u = jnp.maximum(
      jnp.dot(t, w1_ref[...], preferred_element_type=jnp.float32) * a1_ref[...]
      + c1_ref[...], 0.0)
  v = jnp.maximum(
      jnp.dot(u, w2_ref[...], preferred_element_type=jnp.float32) * a2_ref[...]
      + c2_ref[...], 0.0)
  o = jnp.maximum(v * a3_ref[...] + c3_ref[...], 0.0)
  if residual:
    o = o + h
  o_ref[...] = o


def _gin_stage(h, p, eps, W1, b1, g1, be1, W2, b2, g2, be2, bng, bnb, residual):
  a1 = (g1 * ISQ).reshape(1, D)
  c1 = (b1 * g1 * ISQ + be1).reshape(1, D)
  a2 = (g2 * ISQ).reshape(1, D)
  c2 = (b2 * g2 * ISQ + be2).reshape(1, D)
  a3 = (bng * ISQ).reshape(1, D)
  c3 = bnb.reshape(1, D)
  return pl.pallas_call(
      functools.partial(_gin_body, residual=residual),
      grid=(GRID,),
      in_specs=[_row_spec(D), _pair_spec(D), pl.BlockSpec((1, 1), lambda i: (0, 0)),
                _w_spec(), _v_spec(), _v_spec(),
                _w_spec(), _v_spec(), _v_spec(), _v_spec(), _v_spec()],
      out_specs=_row_spec(D),
      out_shape=jax.ShapeDtypeStruct((N, D), jnp.float32),
  )(h, p, eps.reshape(1, 1), W1, a1, c1, W2, a2, c2, a3, c3)


def _sage_body(h_ref, p_ref, dp_ref, wl_ref, wr_ref, a_ref, cb_ref, o_ref):
  h = h_ref[...]
  deg = dp_ref[0, :, 0:1] + dp_ref[1, :, 0:1]
  mean = (p_ref[0] + p_ref[1]) / jnp.maximum(deg, 1.0)
  m = (jnp.dot(mean, wl_ref[...], preferred_element_type=jnp.float32)
       + jnp.dot(h, wr_ref[...], preferred_element_type=jnp.float32))
  o_ref[...] = jnp.maximum(m * a_ref[...] + cb_ref[...], 0.0) + h


def _sage_stage(h, p, dp, Wl, bl, Wr, bng, bnb):
  a = (bng * ISQ).reshape(1, D)
  cb = (bl * bng * ISQ + bnb).reshape(1, D)
  return pl.pallas_call(
      _sage_body,
      grid=(GRID,),
      in_specs=[_row_spec(D), _pair_spec(D), _pair_spec(DEGW),
                _w_spec(), _w_spec(), _v_spec(), _v_spec()],
      out_specs=_row_spec(D),
      out_shape=jax.ShapeDtypeStruct((N, D), jnp.float32),
  )(h, p, dp, Wl, Wr, a, cb)


def _head_body(h1_ref, h2_ref, h3_ref, wa_ref, wb_ref, wc_ref, b1_ref,
               w2_ref, b2_ref, o_ref):
  h4 = jnp.maximum(
      jnp.dot(h1_ref[...], wa_ref[...], preferred_element_type=jnp.float32)
      + jnp.dot(h2_ref[...], wb_ref[...], preferred_element_type=jnp.float32)
      + jnp.dot(h3_ref[...], wc_ref[...], preferred_element_type=jnp.float32)
      + b1_ref[...], 0.0)
  logits = jnp.dot(h4, w2_ref[...], preferred_element_type=jnp.float32) + b2_ref[...]
  col = lax.broadcasted_iota(jnp.int32, (R, D), 1)
  valid = col < C
  lm = jnp.max(jnp.where(valid, logits, -1e30), axis=1, keepdims=True)
  e = jnp.where(valid, jnp.exp(logits - lm), 0.0)
  lse = lm + jnp.log(jnp.sum(e, axis=1, keepdims=True))
  o_ref[...] = logits - lse


def _head_stage(h1, h2, h3, lin1_W, lin1_b, lin2_W, lin2_b):
  Wa = lin1_W[0:D]
  Wb = lin1_W[D:2 * D]
  Wc = lin1_W[2 * D:3 * D]
  b1 = lin1_b.reshape(1, D)
  W2p = jnp.zeros((D, D), jnp.float32).at[:, :C].set(lin2_W)
  b2p = jnp.zeros((1, D), jnp.float32).at[0, :C].set(lin2_b)
  out = pl.pallas_call(
      _head_body,
      grid=(GRID,),
      in_specs=[_row_spec(D), _row_spec(D), _row_spec(D),
                _w_spec(), _w_spec(), _w_spec(), _v_spec(),
                _w_spec(), _v_spec()],
      out_specs=_row_spec(D),
      out_shape=jax.ShapeDtypeStruct((N, D), jnp.float32),
  )(h1, h2, h3, Wa, Wb, Wc, b1, W2p, b2p)
  return out[:, :C]


def kernel(x, edge_index, W_in, b_in, eps1, gin1_W1, gin1_b1, gin1_g1,
           gin1_be1, gin1_W2, gin1_b2, gin1_g2, gin1_be2, bn1_g, bn1_b,
           sage_Wl, sage_bl, sage_Wr, bn2_g, bn2_b, eps2, gin2_W1, gin2_b1,
           gin2_g1, gin2_be1, gin2_W2, gin2_b2, gin2_g2, gin2_be2, bn3_g,
           bn3_b, lin1_W, lin1_b, lin2_W, lin2_b):
  src = edge_index[0]
  dst = edge_index[1]
  zeros_nd = jnp.zeros((N, D), jnp.float32)
  zeros_nw = jnp.zeros((N, DEGW), jnp.float32)

  h = _in_proj(x, W_in, b_in.reshape(1, D))

  agg1, deg = _seg_sum_deg(h, src, dst, zeros_nd, zeros_nw)
  h1 = _gin_stage(h, agg1, eps1, gin1_W1, gin1_b1, gin1_g1, gin1_be1,
                  gin1_W2, gin1_b2, gin1_g2, gin1_be2, bn1_g, bn1_b,
                  residual=False)

  agg2 = _seg_sum(h1, src, dst, zeros_nd)
  h2 = _sage_stage(h1, agg2, deg, sage_Wl, sage_bl, sage_Wr, bn2_g, bn2_b)

  agg3 = _seg_sum(h2, src, dst, zeros_nd)
  h3 = _gin_stage(h2, agg3, eps2, gin2_W1, gin2_b1, gin2_g1, gin2_be1,
                  gin2_W2, gin2_b2, gin2_g2, gin2_be2, bn3_g, bn3_b,
                  residual=True)

  return _head_stage(h1, h2, h3, lin1_W, lin1_b, lin2_W, lin2_b)


# speedup vs baseline: 4.1005x; 4.1005x over previous
import functools
import math

import jax
import jax.numpy as jnp
from jax import lax
from jax.experimental import pallas as pl
from jax.experimental.pallas import tpu as pltpu
from jax.experimental.pallas import tpu_sc as plsc

N = 10000
E = 320000
D = 128
C = 2
BN_EPS = 1e-5
ISQ = float(1.0 / math.sqrt(1.0 + BN_EPS))

NC = 2
NS = 16
NW = NC * NS
EPW = E // NW
CH = 80
NCHUNK = EPW // CH
RPT = 632
RPT_LAST = N - (NS - 1) * RPT
DEGW = 16


def _make_seg_sum():
  mesh = plsc.VectorSubcoreMesh(core_axis_name="c", subcore_axis_name="s",
                                num_cores=NC, num_subcores=NS)
  out_type = jax.ShapeDtypeStruct((NC, N, D), jnp.float32)
  scratch = [
      pltpu.VMEM((CH,), jnp.int32),
      pltpu.VMEM((CH,), jnp.int32),
      pltpu.VMEM((CH, D), jnp.float32),
      pltpu.VMEM_SHARED((N, D), jnp.float32),
      pltpu.SemaphoreType.DMA,
  ]

  def body(h_hbm, src_hbm, dst_hbm, z_hbm, out_hbm,
           src_v, dst_v, rows_v, acc, sem):
    c = lax.axis_index("c")
    s = lax.axis_index("s")
    wid = s * NC + c
    r0 = s * RPT

    @pl.when(s < NS - 1)
    def _():
      pltpu.sync_copy(z_hbm.at[pl.ds(r0, RPT)], acc.at[pl.ds(r0, RPT)])

    @pl.when(s == NS - 1)
    def _():
      pltpu.sync_copy(z_hbm.at[pl.ds(r0, RPT_LAST)], acc.at[pl.ds(r0, RPT_LAST)])

    plsc.subcore_barrier()

    base = wid * EPW

    def chunk(i, _):
      off = base + i * CH
      pltpu.sync_copy(src_hbm.at[pl.ds(off, CH)], src_v)
      pltpu.sync_copy(dst_hbm.at[pl.ds(off, CH)], dst_v)
      pltpu.async_copy(h_hbm.at[src_v], rows_v, sem).wait()
      pltpu.sync_copy(rows_v, acc.at[dst_v], add=True)
      return 0

    lax.fori_loop(0, NCHUNK, chunk, 0)
    plsc.subcore_barrier()

    @pl.when(s < NS - 1)
    def _():
      pltpu.sync_copy(acc.at[pl.ds(r0, RPT)], out_hbm.at[c, pl.ds(r0, RPT)])

    @pl.when(s == NS - 1)
    def _():
      pltpu.sync_copy(acc.at[pl.ds(r0, RPT_LAST)],
                      out_hbm.at[c, pl.ds(r0, RPT_LAST)])

  return pl.kernel(body, out_type=out_type, mesh=mesh, scratch_types=scratch)


def _make_deg():
  mesh = plsc.VectorSubcoreMesh(core_axis_name="c", subcore_axis_name="s",
                                num_cores=NC, num_subcores=NS)
  out_type = jax.ShapeDtypeStruct((NC, N, D), jnp.float32)
  scratch = [
      pltpu.VMEM((CH,), jnp.int32),
      pltpu.VMEM((CH, D), jnp.float32),
      pltpu.VMEM_SHARED((N, D), jnp.float32),
  ]

  def body(dst_hbm, z_hbm, ones_hbm, out_hbm, dst_v, ones_v, acc):
    c = lax.axis_index("c")
    s = lax.axis_index("s")
    wid = s * NC + c
    r0 = s * RPT

    pltpu.sync_copy(ones_hbm, ones_v)

    @pl.when(s < NS - 1)
    def _():
      pltpu.sync_copy(z_hbm.at[pl.ds(r0, RPT)], acc.at[pl.ds(r0, RPT)])

    @pl.when(s == NS - 1)
    def _():
      pltpu.sync_copy(z_hbm.at[pl.ds(r0, RPT_LAST)], acc.at[pl.ds(r0, RPT_LAST)])

    plsc.subcore_barrier()

    base = wid * EPW

    def chunk(i, _):
      off = base + i * CH
      pltpu.sync_copy(dst_hbm.at[pl.ds(off, CH)], dst_v)
      pltpu.sync_copy(ones_v, acc.at[dst_v], add=True)
      return 0

    lax.fori_loop(0, NCHUNK, chunk, 0)
    plsc.subcore_barrier()

    @pl.when(s < NS - 1)
    def _():
      pltpu.sync_copy(acc.at[pl.ds(r0, RPT)], out_hbm.at[c, pl.ds(r0, RPT)])

    @pl.when(s == NS - 1)
    def _():
      pltpu.sync_copy(acc.at[pl.ds(r0, RPT_LAST)],
                      out_hbm.at[c, pl.ds(r0, RPT_LAST)])

  return pl.kernel(body, out_type=out_type, mesh=mesh, scratch_types=scratch)


@functools.lru_cache(maxsize=None)
def _seg_sum_call():
  return _make_seg_sum()


@functools.lru_cache(maxsize=None)
def _deg_call():
  return _make_deg()


R = 2000
GRID = N // R

def _row_spec(w):
  return pl.BlockSpec((R, w), lambda i: (i, 0))

def _pair_spec(w):
  return pl.BlockSpec((NC, R, w), lambda i: (0, i, 0))

def _w_spec():
  return pl.BlockSpec((D, D), lambda i: (0, 0))

def _v_spec():
  return pl.BlockSpec((1, D), lambda i: (0, 0))


def _in_proj_body(x_ref, w_ref, b_ref, o_ref):
  o_ref[...] = jnp.dot(x_ref[...], w_ref[...],
                       preferred_element_type=jnp.float32) + b_ref[...]


def _in_proj(x, w, b):
  return pl.pallas_call(
      _in_proj_body,
      grid=(GRID,),
      in_specs=[_row_spec(D), _w_spec(), _v_spec()],
      out_specs=_row_spec(D),
      out_shape=jax.ShapeDtypeStruct((N, D), jnp.float32),
  )(x, w, b)


def _gin_body(h_ref, p_ref, eps_ref, w1_ref, a1_ref, c1_ref,
              w2_ref, a2_ref, c2_ref, a3_ref, c3_ref, o_ref, *, residual):
  h = h_ref[...]
  t = (1.0 + eps_ref[0, 0]) * h + p_ref[0] + p_ref[1]
  u = jnp.maximum(
      jnp.dot(t, w1_ref[...], preferred_element_type=jnp.float32) * a1_ref[...]
      + c1_ref[...], 0.0)
  v = jnp.maximum(
      jnp.dot(u, w2_ref[...], preferred_element_type=jnp.float32) * a2_ref[...]
      + c2_ref[...], 0.0)
  o = jnp.maximum(v * a3_ref[...] + c3_ref[...], 0.0)
  if residual:
    o = o + h
  o_ref[...] = o


def _gin_stage(h, p, eps, W1, b1, g1, be1, W2, b2, g2, be2, bng, bnb, residual):
  a1 = (g1 * ISQ).reshape(1, D)
  c1 = (b1 * g1 * ISQ + be1).reshape(1, D)
  a2 = (g2 * ISQ).reshape(1, D)
  c2 = (b2 * g2 * ISQ + be2).reshape(1, D)
  a3 = (bng * ISQ).reshape(1, D)
  c3 = bnb.reshape(1, D)
  return pl.pallas_call(
      functools.partial(_gin_body, residual=residual),
      grid=(GRID,),
      in_specs=[_row_spec(D), _pair_spec(D), pl.BlockSpec((1, 1), lambda i: (0, 0)),
                _w_spec(), _v_spec(), _v_spec(),
                _w_spec(), _v_spec(), _v_spec(), _v_spec(), _v_spec()],
      out_specs=_row_spec(D),
      out_shape=jax.ShapeDtypeStruct((N, D), jnp.float32),
  )(h, p, eps.reshape(1, 1), W1, a1, c1, W2, a2, c2, a3, c3)


def _sage_body(h_ref, p_ref, dp_ref, wl_ref, wr_ref, a_ref, cb_ref, o_ref):
  h = h_ref[...]
  deg = dp_ref[0, :, 0:1] + dp_ref[1, :, 0:1]
  mean = (p_ref[0] + p_ref[1]) / jnp.maximum(deg, 1.0)
  m = (jnp.dot(mean, wl_ref[...], preferred_element_type=jnp.float32)
       + jnp.dot(h, wr_ref[...], preferred_element_type=jnp.float32))
  o_ref[...] = jnp.maximum(m * a_ref[...] + cb_ref[...], 0.0) + h


def _sage_stage(h, p, dp, Wl, bl, Wr, bng, bnb):
  a = (bng * ISQ).reshape(1, D)
  cb = (bl * bng * ISQ + bnb).reshape(1, D)
  return pl.pallas_call(
      _sage_body,
      grid=(GRID,),
      in_specs=[_row_spec(D), _pair_spec(D), _pair_spec(D),
                _w_spec(), _w_spec(), _v_spec(), _v_spec()],
      out_specs=_row_spec(D),
      out_shape=jax.ShapeDtypeStruct((N, D), jnp.float32),
  )(h, p, dp, Wl, Wr, a, cb)


def _head_body(h1_ref, h2_ref, h3_ref, wa_ref, wb_ref, wc_ref, b1_ref,
               w2_ref, b2_ref, o_ref):
  h4 = jnp.maximum(
      jnp.dot(h1_ref[...], wa_ref[...], preferred_element_type=jnp.float32)
      + jnp.dot(h2_ref[...], wb_ref[...], preferred_element_type=jnp.float32)
      + jnp.dot(h3_ref[...], wc_ref[...], preferred_element_type=jnp.float32)
      + b1_ref[...], 0.0)
  logits = jnp.dot(h4, w2_ref[...], preferred_element_type=jnp.float32) + b2_ref[...]
  col = lax.broadcasted_iota(jnp.int32, (R, D), 1)
  valid = col < C
  lm = jnp.max(jnp.where(valid, logits, -1e30), axis=1, keepdims=True)
  e = jnp.where(valid, jnp.exp(logits - lm), 0.0)
  lse = lm + jnp.log(jnp.sum(e, axis=1, keepdims=True))
  o_ref[...] = logits - lse


def _head_stage(h1, h2, h3, lin1_W, lin1_b, lin2_W, lin2_b):
  Wa = lin1_W[0:D]
  Wb = lin1_W[D:2 * D]
  Wc = lin1_W[2 * D:3 * D]
  b1 = lin1_b.reshape(1, D)
  W2p = jnp.zeros((D, D), jnp.float32).at[:, :C].set(lin2_W)
  b2p = jnp.zeros((1, D), jnp.float32).at[0, :C].set(lin2_b)
  out = pl.pallas_call(
      _head_body,
      grid=(GRID,),
      in_specs=[_row_spec(D), _row_spec(D), _row_spec(D),
                _w_spec(), _w_spec(), _w_spec(), _v_spec(),
                _w_spec(), _v_spec()],
      out_specs=_row_spec(D),
      out_shape=jax.ShapeDtypeStruct((N, D), jnp.float32),
  )(h1, h2, h3, Wa, Wb, Wc, b1, W2p, b2p)
  return out[:, :C]


def kernel(x, edge_index, W_in, b_in, eps1, gin1_W1, gin1_b1, gin1_g1,
           gin1_be1, gin1_W2, gin1_b2, gin1_g2, gin1_be2, bn1_g, bn1_b,
           sage_Wl, sage_bl, sage_Wr, bn2_g, bn2_b, eps2, gin2_W1, gin2_b1,
           gin2_g1, gin2_be1, gin2_W2, gin2_b2, gin2_g2, gin2_be2, bn3_g,
           bn3_b, lin1_W, lin1_b, lin2_W, lin2_b):
  src = edge_index[0]
  dst = edge_index[1]
  zeros_nd = jnp.zeros((N, D), jnp.float32)
  ones_ch = jnp.ones((CH, D), jnp.float32)

  deg = _deg_call()(dst, zeros_nd, ones_ch)

  h = _in_proj(x, W_in, b_in.reshape(1, D))

  agg1 = _seg_sum_call()(h, src, dst, zeros_nd)
  h1 = _gin_stage(h, agg1, eps1, gin1_W1, gin1_b1, gin1_g1, gin1_be1,
                  gin1_W2, gin1_b2, gin1_g2, gin1_be2, bn1_g, bn1_b,
                  residual=False)

  agg2 = _seg_sum_call()(h1, src, dst, zeros_nd)
  h2 = _sage_stage(h1, agg2, deg, sage_Wl, sage_bl, sage_Wr, bn2_g, bn2_b)

  agg3 = _seg_sum_call()(h2, src, dst, zeros_nd)
  h3 = _gin_stage(h2, agg3, eps2, gin2_W1, gin2_b1, gin2_g1, gin2_be1,
                  gin2_W2, gin2_b2, gin2_g2, gin2_be2, bn3_g, bn3_b,
                  residual=True)

  return _head_stage(h1, h2, h3, lin1_W, lin1_b, lin2_W, lin2_b)


# speedup vs baseline: 4.5862x; 1.1185x over previous
import functools
import math

import jax
import jax.numpy as jnp
from jax import lax
from jax.experimental import pallas as pl
from jax.experimental.pallas import tpu as pltpu
from jax.experimental.pallas import tpu_sc as plsc

N = 10000
E = 320000
D = 128
C = 2
BN_EPS = 1e-5
ISQ = float(1.0 / math.sqrt(1.0 + BN_EPS))

NC = 2
NS = 16
NW = NC * NS
EPW = E // NW
CH = 40
NCHUNK = EPW // CH
RPT = 632
RPT_LAST = N - (NS - 1) * RPT
DEGW = 16


K = 5
NSUP = NCHUNK // K


def _zero_init(z_hbm, acc, s):
  @pl.when(s < NS - 1)
  def _():
    pltpu.sync_copy(z_hbm.at[pl.ds(s * RPT, RPT)], acc.at[pl.ds(s * RPT, RPT)])

  @pl.when(s == NS - 1)
  def _():
    pltpu.sync_copy(z_hbm.at[pl.ds(s * RPT, RPT_LAST)],
                    acc.at[pl.ds(s * RPT, RPT_LAST)])


def _copy_out(acc, out_hbm, c, s):
  @pl.when(s < NS - 1)
  def _():
    pltpu.sync_copy(acc.at[pl.ds(s * RPT, RPT)],
                    out_hbm.at[c, pl.ds(s * RPT, RPT)])

  @pl.when(s == NS - 1)
  def _():
    pltpu.sync_copy(acc.at[pl.ds(s * RPT, RPT_LAST)],
                    out_hbm.at[c, pl.ds(s * RPT, RPT_LAST)])


def _make_seg_sum():
  mesh = plsc.VectorSubcoreMesh(core_axis_name="c", subcore_axis_name="s",
                                num_cores=NC, num_subcores=NS)
  out_type = jax.ShapeDtypeStruct((NC, N, D), jnp.float32)
  scratch = (
      [pltpu.VMEM((CH,), jnp.int32) for _ in range(K)]
      + [pltpu.VMEM((CH,), jnp.int32) for _ in range(K)]
      + [pltpu.VMEM((CH, D), jnp.float32) for _ in range(K)]
      + [pltpu.VMEM_SHARED((N, D), jnp.float32)]
      + [pltpu.SemaphoreType.DMA for _ in range(2 * K)]
  )

  def body(h_hbm, src_hbm, dst_hbm, z_hbm, out_hbm, *scr):
    srcs = scr[0:K]
    dsts = scr[K:2 * K]
    rows = scr[2 * K:3 * K]
    acc = scr[3 * K]
    gsems = scr[3 * K + 1:4 * K + 1]
    ssems = scr[4 * K + 1:5 * K + 1]

    c = lax.axis_index("c")
    s = lax.axis_index("s")
    wid = s * NC + c
    base = wid * EPW

    _zero_init(z_hbm, acc, s)
    plsc.subcore_barrier()

    def superstep(g, _):
      for j in range(K):
        @pl.when(g > 0)
        def _(j=j):
          pltpu.make_async_copy(rows[j], acc.at[dsts[j]], ssems[j]).wait()
        off = base + (g * K + j) * CH
        pltpu.sync_copy(src_hbm.at[pl.ds(off, CH)], srcs[j])
        pltpu.sync_copy(dst_hbm.at[pl.ds(off, CH)], dsts[j])
        pltpu.async_copy(h_hbm.at[srcs[j]], rows[j], gsems[j])
      for j in range(K):
        pltpu.make_async_copy(h_hbm.at[srcs[j]], rows[j], gsems[j]).wait()
        pltpu.async_copy(rows[j], acc.at[dsts[j]], ssems[j], add=True)
      return 0

    lax.fori_loop(0, NSUP, superstep, 0)
    for j in range(K):
      pltpu.make_async_copy(rows[j], acc.at[dsts[j]], ssems[j]).wait()
    plsc.subcore_barrier()
    _copy_out(acc, out_hbm, c, s)

  return pl.kernel(body, out_type=out_type, mesh=mesh, scratch_types=scratch)


def _make_deg():
  mesh = plsc.VectorSubcoreMesh(core_axis_name="c", subcore_axis_name="s",
                                num_cores=NC, num_subcores=NS)
  out_type = jax.ShapeDtypeStruct((NC, N, D), jnp.float32)
  scratch = (
      [pltpu.VMEM((CH,), jnp.int32) for _ in range(K)]
      + [pltpu.VMEM((CH, D), jnp.float32)]
      + [pltpu.VMEM_SHARED((N, D), jnp.float32)]
      + [pltpu.SemaphoreType.DMA for _ in range(K)]
  )

  def body(dst_hbm, z_hbm, ones_hbm, out_hbm, *scr):
    dsts = scr[0:K]
    ones_v = scr[K]
    acc = scr[K + 1]
    ssems = scr[K + 2:2 * K + 2]

    c = lax.axis_index("c")
    s = lax.axis_index("s")
    wid = s * NC + c
    base = wid * EPW

    pltpu.sync_copy(ones_hbm, ones_v)
    _zero_init(z_hbm, acc, s)
    plsc.subcore_barrier()

    def superstep(g, _):
      for j in range(K):
        @pl.when(g > 0)
        def _(j=j):
          pltpu.make_async_copy(ones_v, acc.at[dsts[j]], ssems[j]).wait()
        off = base + (g * K + j) * CH
        pltpu.sync_copy(dst_hbm.at[pl.ds(off, CH)], dsts[j])
        pltpu.async_copy(ones_v, acc.at[dsts[j]], ssems[j], add=True)
      return 0

    lax.fori_loop(0, NSUP, superstep, 0)
    for j in range(K):
      pltpu.make_async_copy(ones_v, acc.at[dsts[j]], ssems[j]).wait()
    plsc.subcore_barrier()
    _copy_out(acc, out_hbm, c, s)

  return pl.kernel(body, out_type=out_type, mesh=mesh, scratch_types=scratch)


@functools.lru_cache(maxsize=None)
def _seg_sum_call():
  return _make_seg_sum()


@functools.lru_cache(maxsize=None)
def _deg_call():
  return _make_deg()


R = 2000
GRID = N // R

def _row_spec(w):
  return pl.BlockSpec((R, w), lambda i: (i, 0))

def _pair_spec(w):
  return pl.BlockSpec((NC, R, w), lambda i: (0, i, 0))

def _w_spec():
  return pl.BlockSpec((D, D), lambda i: (0, 0))

def _v_spec():
  return pl.BlockSpec((1, D), lambda i: (0, 0))


def _in_proj_body(x_ref, w_ref, b_ref, o_ref):
  o_ref[...] = jnp.dot(x_ref[...], w_ref[...],
                       preferred_element_type=jnp.float32) + b_ref[...]


def _in_proj(x, w, b):
  return pl.pallas_call(
      _in_proj_body,
      grid=(GRID,),
      in_specs=[_row_spec(D), _w_spec(), _v_spec()],
      out_specs=_row_spec(D),
      out_shape=jax.ShapeDtypeStruct((N, D), jnp.float32),
  )(x, w, b)


def _gin_body(h_ref, p_ref, eps_ref, w1_ref, a1_ref, c1_ref,
              w2_ref, a2_ref, c2_ref, a3_ref, c3_ref, o_ref, *, residual):
  h = h_ref[...]
  t = (1.0 + eps_ref[0, 0]) * h + p_ref[0] + p_ref[1]
  u = jnp.maximum(
      jnp.dot(t, w1_ref[...], preferred_element_type=jnp.float32) * a1_ref[...]
      + c1_ref[...], 0.0)
  v = jnp.maximum(
      jnp.dot(u, w2_ref[...], preferred_element_type=jnp.float32) * a2_ref[...]
      + c2_ref[...], 0.0)
  o = jnp.maximum(v * a3_ref[...] + c3_ref[...], 0.0)
  if residual:
    o = o + h
  o_ref[...] = o


def _gin_stage(h, p, eps, W1, b1, g1, be1, W2, b2, g2, be2, bng, bnb, residual):
  a1 = (g1 * ISQ).reshape(1, D)
  c1 = (b1 * g1 * ISQ + be1).reshape(1, D)
  a2 = (g2 * ISQ).reshape(1, D)
  c2 = (b2 * g2 * ISQ + be2).reshape(1, D)
  a3 = (bng * ISQ).reshape(1, D)
  c3 = bnb.reshape(1, D)
  return pl.pallas_call(
      functools.partial(_gin_body, residual=residual),
      grid=(GRID,),
      in_specs=[_row_spec(D), _pair_spec(D), pl.BlockSpec((1, 1), lambda i: (0, 0)),
                _w_spec(), _v_spec(), _v_spec(),
                _w_spec(), _v_spec(), _v_spec(), _v_spec(), _v_spec()],
      out_specs=_row_spec(D),
      out_shape=jax.ShapeDtypeStruct((N, D), jnp.float32),
  )(h, p, eps.reshape(1, 1), W1, a1, c1, W2, a2, c2, a3, c3)


def _sage_body(h_ref, p_ref, dp_ref, wl_ref, wr_ref, a_ref, cb_ref, o_ref):
  h = h_ref[...]
  deg = dp_ref[0, :, 0:1] + dp_ref[1, :, 0:1]
  mean = (p_ref[0] + p_ref[1]) / jnp.maximum(deg, 1.0)
  m = (jnp.dot(mean, wl_ref[...], preferred_element_type=jnp.float32)
       + jnp.dot(h, wr_ref[...], preferred_element_type=jnp.float32))
  o_ref[...] = jnp.maximum(m * a_ref[...] + cb_ref[...], 0.0) + h


def _sage_stage(h, p, dp, Wl, bl, Wr, bng, bnb):
  a = (bng * ISQ).reshape(1, D)
  cb = (bl * bng * ISQ + bnb).reshape(1, D)
  return pl.pallas_call(
      _sage_body,
      grid=(GRID,),
      in_specs=[_row_spec(D), _pair_spec(D), _pair_spec(D),
                _w_spec(), _w_spec(), _v_spec(), _v_spec()],
      out_specs=_row_spec(D),
      out_shape=jax.ShapeDtypeStruct((N, D), jnp.float32),
  )(h, p, dp, Wl, Wr, a, cb)


def _head_body(h1_ref, h2_ref, h3_ref, wa_ref, wb_ref, wc_ref, b1_ref,
               w2_ref, b2_ref, o_ref):
  h4 = jnp.maximum(
      jnp.dot(h1_ref[...], wa_ref[...], preferred_element_type=jnp.float32)
      + jnp.dot(h2_ref[...], wb_ref[...], preferred_element_type=jnp.float32)
      + jnp.dot(h3_ref[...], wc_ref[...], preferred_element_type=jnp.float32)
      + b1_ref[...], 0.0)
  logits = jnp.dot(h4, w2_ref[...], preferred_element_type=jnp.float32) + b2_ref[...]
  col = lax.broadcasted_iota(jnp.int32, (R, D), 1)
  valid = col < C
  lm = jnp.max(jnp.where(valid, logits, -1e30), axis=1, keepdims=True)
  e = jnp.where(valid, jnp.exp(logits - lm), 0.0)
  lse = lm + jnp.log(jnp.sum(e, axis=1, keepdims=True))
  o_ref[...] = logits - lse


def _head_stage(h1, h2, h3, lin1_W, lin1_b, lin2_W, lin2_b):
  Wa = lin1_W[0:D]
  Wb = lin1_W[D:2 * D]
  Wc = lin1_W[2 * D:3 * D]
  b1 = lin1_b.reshape(1, D)
  W2p = jnp.zeros((D, D), jnp.float32).at[:, :C].set(lin2_W)
  b2p = jnp.zeros((1, D), jnp.float32).at[0, :C].set(lin2_b)
  out = pl.pallas_call(
      _head_body,
      grid=(GRID,),
      in_specs=[_row_spec(D), _row_spec(D), _row_spec(D),
                _w_spec(), _w_spec(), _w_spec(), _v_spec(),
                _w_spec(), _v_spec()],
      out_specs=_row_spec(D),
      out_shape=jax.ShapeDtypeStruct((N, D), jnp.float32),
  )(h1, h2, h3, Wa, Wb, Wc, b1, W2p, b2p)
  return out[:, :C]


def kernel(x, edge_index, W_in, b_in, eps1, gin1_W1, gin1_b1, gin1_g1,
           gin1_be1, gin1_W2, gin1_b2, gin1_g2, gin1_be2, bn1_g, bn1_b,
           sage_Wl, sage_bl, sage_Wr, bn2_g, bn2_b, eps2, gin2_W1, gin2_b1,
           gin2_g1, gin2_be1, gin2_W2, gin2_b2, gin2_g2, gin2_be2, bn3_g,
           bn3_b, lin1_W, lin1_b, lin2_W, lin2_b):
  src = edge_index[0]
  dst = edge_index[1]
  zeros_nd = jnp.zeros((N, D), jnp.float32)
  ones_ch = jnp.ones((CH, D), jnp.float32)

  deg = _deg_call()(dst, zeros_nd, ones_ch)

  h = _in_proj(x, W_in, b_in.reshape(1, D))

  agg1 = _seg_sum_call()(h, src, dst, zeros_nd)
  h1 = _gin_stage(h, agg1, eps1, gin1_W1, gin1_b1, gin1_g1, gin1_be1,
                  gin1_W2, gin1_b2, gin1_g2, gin1_be2, bn1_g, bn1_b,
                  residual=False)

  agg2 = _seg_sum_call()(h1, src, dst, zeros_nd)
  h2 = _sage_stage(h1, agg2, deg, sage_Wl, sage_bl, sage_Wr, bn2_g, bn2_b)

  agg3 = _seg_sum_call()(h2, src, dst, zeros_nd)
  h3 = _gin_stage(h2, agg3, eps2, gin2_W1, gin2_b1, gin2_g1, gin2_be1,
                  gin2_W2, gin2_b2, gin2_g2, gin2_be2, bn3_g, bn3_b,
                  residual=True)

  return _head_stage(h1, h2, h3, lin1_W, lin1_b, lin2_W, lin2_b)
